# baseline (device time: 132504 ns/iter reference)
import jax
import jax.numpy as jnp
from jax import lax
from jax.experimental import pallas as pl
from jax.experimental.pallas import tpu as pltpu

N_DEV = 16
ROWS = 4096
D = 1024
NBITS = 13


def _a2av_body(cnt_ref, lo_ref, cntv_ref, lorder_ref, x_ref,
               out_ref, xs_ref, c_ref, c_smem, roff_ref,
               csend_sems, crecv_sems, copy_sem, send_sems, recv_sems):
    me = lax.axis_index("i")

    barrier = pltpu.get_barrier_semaphore()
    for k in range(1, N_DEV):
        pl.semaphore_signal(barrier, inc=1, device_id=((me + k) % N_DEV,),
                            device_id_type=pl.DeviceIdType.MESH)
    pl.semaphore_wait(barrier, N_DEV - 1)

    c_ref[pl.ds(me, 1)] = cntv_ref[...]
    count_rdmas = []
    for k in range(1, N_DEV):
        peer = (me + k) % N_DEV
        r = pltpu.make_async_remote_copy(
            src_ref=cntv_ref,
            dst_ref=c_ref.at[pl.ds(me, 1)],
            send_sem=csend_sems.at[k - 1],
            recv_sem=crecv_sems.at[me],
            device_id=(peer,),
            device_id_type=pl.DeviceIdType.MESH,
        )
        r.start()
        count_rdmas.append(r)

    for s in range(N_DEV):
        @pl.when(s != me)
        def _(s=s):
            rw = pltpu.make_async_remote_copy(
                src_ref=cntv_ref,
                dst_ref=c_ref.at[pl.ds(s, 1)],
                send_sem=csend_sems.at[0],
                recv_sem=crecv_sems.at[s],
                device_id=(s,),
                device_id_type=pl.DeviceIdType.MESH,
            )
            rw.wait_recv()
    for r in count_rdmas:
        r.wait_send()

    cp = pltpu.make_async_copy(c_ref, c_smem, copy_sem)
    cp.start()
    cp.wait()

    for d in range(N_DEV):
        acc = jnp.int32(0)
        for s in range(N_DEV):
            acc = acc + jnp.where(s < me, c_smem[s, 0, d], 0)
        roff_ref[d] = acc

    for t in range(N_DEV - 1):
        d = (me + 1 + t) % N_DEV
        c = cnt_ref[d]
        base_s = lo_ref[d]
        base_r = roff_ref[d]

        def group_row(k, _, base_s=base_s):
            idx = lorder_ref[base_s + k]
            xs_ref[pl.ds(base_s + k, 1)] = x_ref[pl.ds(idx, 1)]
            return 0

        lax.fori_loop(0, c, group_row, 0)

        cur_s = base_s
        cur_r = base_r
        for b in range(NBITS - 1, -1, -1):
            sz = 1 << b
            has = ((c >> b) & 1) == 1

            @pl.when(has)
            def _(d=d, b=b, sz=sz, cur_s=cur_s, cur_r=cur_r):
                r = pltpu.make_async_remote_copy(
                    src_ref=xs_ref.at[pl.ds(cur_s, sz)],
                    dst_ref=out_ref.at[pl.ds(cur_r, sz)],
                    send_sem=send_sems.at[d, b],
                    recv_sem=recv_sems.at[me, b],
                    device_id=(d,),
                    device_id_type=pl.DeviceIdType.MESH,
                )
                r.start()

            inc = jnp.where(has, sz, 0)
            cur_s = cur_s + inc
            cur_r = cur_r + inc

    base_own_s = lo_ref[me]
    base_own_r = roff_ref[me]

    def own_row(k, _):
        idx = lorder_ref[base_own_s + k]
        out_ref[pl.ds(base_own_r + k, 1)] = x_ref[pl.ds(idx, 1)]
        return 0

    lax.fori_loop(0, cnt_ref[me], own_row, 0)

    for t in range(N_DEV - 1):
        d = (me + 1 + t) % N_DEV
        c = cnt_ref[d]
        for b in range(NBITS - 1, -1, -1):
            sz = 1 << b
            has = ((c >> b) & 1) == 1

            @pl.when(has)
            def _(d=d, b=b, sz=sz):
                r = pltpu.make_async_remote_copy(
                    src_ref=xs_ref.at[pl.ds(0, sz)],
                    dst_ref=out_ref.at[pl.ds(0, sz)],
                    send_sem=send_sems.at[d, b],
                    recv_sem=recv_sems.at[d, b],
                    device_id=(d,),
                    device_id_type=pl.DeviceIdType.MESH,
                )
                r.wait_send()

    for s in range(N_DEV):
        rc = c_smem[s, 0, me]
        for b in range(NBITS - 1, -1, -1):
            sz = 1 << b
            has = ((rc >> b) & 1) == 1

            @pl.when(has & (me != s))
            def _(s=s, b=b, sz=sz):
                r = pltpu.make_async_remote_copy(
                    src_ref=xs_ref.at[pl.ds(0, sz)],
                    dst_ref=out_ref.at[pl.ds(0, sz)],
                    send_sem=send_sems.at[s, b],
                    recv_sem=recv_sems.at[s, b],
                    device_id=(s,),
                    device_id_type=pl.DeviceIdType.MESH,
                )
                r.wait_recv()


def kernel(x, dest):
    d32 = dest.astype(jnp.int32)

    sd, lorder = lax.sort(
        (d32, jnp.arange(ROWS, dtype=jnp.int32)), num_keys=1, is_stable=True)
    lo = jnp.searchsorted(sd, jnp.arange(N_DEV, dtype=jnp.int32)).astype(
        jnp.int32)
    cnt = (jnp.append(lo[1:], jnp.int32(ROWS)) - lo).astype(jnp.int32)

    x3 = x.astype(jnp.bfloat16).reshape(ROWS, 8, D // 8)
    out = pl.pallas_call(
        _a2av_body,
        out_shape=jax.ShapeDtypeStruct((ROWS, 8, D // 8), jnp.bfloat16),
        in_specs=[
            pl.BlockSpec(memory_space=pltpu.SMEM),
            pl.BlockSpec(memory_space=pltpu.SMEM),
            pl.BlockSpec(memory_space=pltpu.VMEM),
            pl.BlockSpec(memory_space=pltpu.SMEM),
            pl.BlockSpec(memory_space=pltpu.VMEM),
        ],
        out_specs=pl.BlockSpec(memory_space=pltpu.VMEM),
        scratch_shapes=[
            pltpu.VMEM((ROWS, 8, D // 8), jnp.bfloat16),
            pltpu.VMEM((N_DEV, 1, N_DEV), jnp.int32),
            pltpu.SMEM((N_DEV, 1, N_DEV), jnp.int32),
            pltpu.SMEM((N_DEV,), jnp.int32),
            pltpu.SemaphoreType.DMA((N_DEV - 1,)),
            pltpu.SemaphoreType.DMA((N_DEV,)),
            pltpu.SemaphoreType.DMA,
            pltpu.SemaphoreType.DMA((N_DEV, NBITS)),
            pltpu.SemaphoreType.DMA((N_DEV, NBITS)),
        ],
        compiler_params=pltpu.CompilerParams(collective_id=0),
    )(cnt, lo, cnt.reshape(1, 1, N_DEV), lorder, x3)
    return out.reshape(ROWS, D)
